# Initial kernel scaffold; baseline (speedup 1.0000x reference)
import jax
import jax.numpy as jnp
from jax.experimental import pallas as pl


def kernel(x, values, b, rows, cols):
    raise NotImplementedError("write your pallas kernel here")


# speedup vs baseline: 59.7274x; 59.7274x over previous
import jax
import jax.numpy as jnp
from jax import lax
from jax.experimental import pallas as pl
from jax.experimental.pallas import tpu as pltpu
from jax.experimental.pallas import tpu_sc as plsc

_T = 8192
_NB = 512
_B = 8
_L = 16
_NTILES = 32
_EDGES_PER_TILE = _T // _NTILES
_GROUPS = _EDGES_PER_TILE // _L
_JT = 4


def _sc_body(x3_hbm, vals_hbm, ind1_hbm, ind0g_hbm, binit_hbm, out_hbm,
             x3l, vbuf, contrib, ind1b, ind0b, out3):
    c = lax.axis_index("c")
    s = lax.axis_index("s")
    wid = c * 16 + s

    pltpu.sync_copy(x3_hbm, x3l)
    pltpu.sync_copy(ind1_hbm.at[pl.ds(wid * _EDGES_PER_TILE, _EDGES_PER_TILE)],
                    ind1b)
    pltpu.sync_copy(ind0g_hbm.at[pl.ds(wid * _GROUPS, _GROUPS)], ind0b)

    pltpu.sync_copy(binit_hbm.at[c, pl.ds(s * 32, 32)], out3.at[pl.ds(s * 32, 32)])
    plsc.subcore_barrier()

    iota = lax.iota(jnp.int32, _L)
    iota_v = iota * 256

    def group_body(g, carry):
        base_e = wid * _EDGES_PER_TILE + g * _L
        pltpu.sync_copy(vals_hbm.at[pl.ds(base_e * 256, _L * 256)], vbuf)
        ind1v = ind1b[pl.ds(g * _L, _L)]
        rowb = ind1v * 128

        def jt_body(jt, inner):
            jbase = jt * _JT
            acc = [[None] * _B for _ in range(_JT)]
            for i in range(_L):
                vv = [plsc.load_gather(vbuf, [iota_v + (i * 16 + jbase + jp)])
                      for jp in range(_JT)]
                xv = [plsc.load_gather(x3l, [rowb + (i * 8 + bb)])
                      for bb in range(_B)]
                for jp in range(_JT):
                    for bb in range(_B):
                        prod = vv[jp] * xv[bb]
                        if i == 0:
                            acc[jp][bb] = prod
                        else:
                            acc[jp][bb] = acc[jp][bb] + prod
            for jp in range(_JT):
                for bb in range(_B):
                    col = bb * 16 + jbase + jp
                    plsc.store_scatter(contrib,
                                       [iota, jnp.broadcast_to(col, (_L,))],
                                       acc[jp][bb])
            return inner

        lax.fori_loop(0, _L // _JT, jt_body, 0)
        pltpu.sync_copy(contrib, out3.at[ind0b.at[g]], add=True)
        return carry

    lax.fori_loop(0, _GROUPS, group_body, 0)
    plsc.subcore_barrier()
    pltpu.sync_copy(out3.at[pl.ds(s * 32, 32)],
                    out_hbm.at[c, pl.ds(s * 32, 32)])


_KERNEL = pl.kernel(
    _sc_body,
    out_type=jax.ShapeDtypeStruct((2, _NB, 128), jnp.float32),
    mesh=plsc.VectorSubcoreMesh(core_axis_name="c", subcore_axis_name="s"),
    compiler_params=pltpu.CompilerParams(needs_layout_passes=False),
    scratch_types=[
        pltpu.VMEM((_NB * 128,), jnp.float32),
        pltpu.VMEM((_L * 256,), jnp.float32),
        pltpu.VMEM((_L, 128), jnp.float32),
        pltpu.VMEM((_EDGES_PER_TILE,), jnp.int32),
        pltpu.VMEM((_GROUPS, _L), jnp.int32),
        pltpu.VMEM_SHARED((_NB, 128), jnp.float32),
    ],
)


@jax.jit
def kernel(x, values, b, rows, cols):
    ind0 = (rows[:: _L * _L] // _L).astype(jnp.int32)
    ind1 = (cols[:: _L * _L] // _L).astype(jnp.int32)
    ind0g = ind0.reshape(_T // _L, _L)

    x3 = x.reshape(_B, _NB, _L).transpose(1, 2, 0).reshape(-1)

    b0 = jnp.broadcast_to(b.reshape(_NB, 1, _L), (_NB, _B, _L)).reshape(_NB, 128)
    binit = jnp.stack([b0, jnp.zeros_like(b0)])

    outp = _KERNEL(x3, values, ind1, ind0g, binit)
    out = outp[0] + outp[1]
    out = out.reshape(_NB, _B, _L).transpose(1, 0, 2).reshape(_B, _NB * _L, 1)
    return out


# speedup vs baseline: 97.4787x; 1.6321x over previous
import jax
import jax.numpy as jnp
from jax import lax
from jax.experimental import pallas as pl
from jax.experimental.pallas import tpu as pltpu
from jax.experimental.pallas import tpu_sc as plsc

_T = 8192
_NB = 512
_B = 8
_L = 16
_NTILES = 32
_EDGES_PER_TILE = _T // _NTILES
_GROUPS = _EDGES_PER_TILE // _L
_JT = 4


def _sc_body(x3_hbm, vals_hbm, ind1_hbm, ind0g_hbm, binit_hbm, out_hbm,
             x3l, vbuf, contrib, ind1b, ind0b, xstage, out3):
    c = lax.axis_index("c")
    s = lax.axis_index("s")
    wid = c * 16 + s

    pltpu.sync_copy(x3_hbm, x3l)
    pltpu.sync_copy(ind1_hbm.at[pl.ds(wid * _EDGES_PER_TILE, _EDGES_PER_TILE)],
                    ind1b)
    pltpu.sync_copy(ind0g_hbm.at[pl.ds(wid * _GROUPS, _GROUPS)], ind0b)

    pltpu.sync_copy(binit_hbm.at[c, pl.ds(s * 32, 32)], out3.at[pl.ds(s * 32, 32)])
    plsc.subcore_barrier()

    iota = lax.iota(jnp.int32, _L)
    iota_v = iota * 256

    def group_body(g, carry):
        base_e = wid * _EDGES_PER_TILE + g * _L
        pltpu.sync_copy(vals_hbm.at[pl.ds(base_e * 256, _L * 256)], vbuf)
        ind1v = ind1b[pl.ds(g * _L, _L)]
        rowb = ind1v * 128

        for i in range(_L):
            for bb in range(_B):
                xv = plsc.load_gather(x3l, [rowb + (i * 8 + bb)])
                xstage[pl.ds((i * 8 + bb) * _L, _L)] = xv

        def jt_body(jt, inner):
            jbase = jt * _JT
            acc = [[None] * _B for _ in range(_JT)]
            for i in range(_L):
                vv = [plsc.load_gather(vbuf, [iota_v + (i * 16 + jbase + jp)])
                      for jp in range(_JT)]
                xv = [xstage[pl.ds((i * 8 + bb) * _L, _L)]
                      for bb in range(_B)]
                for jp in range(_JT):
                    for bb in range(_B):
                        prod = vv[jp] * xv[bb]
                        if i == 0:
                            acc[jp][bb] = prod
                        else:
                            acc[jp][bb] = acc[jp][bb] + prod
            for jp in range(_JT):
                for bb in range(_B):
                    col = bb * 16 + jbase + jp
                    plsc.store_scatter(contrib,
                                       [iota, jnp.broadcast_to(col, (_L,))],
                                       acc[jp][bb])
            return inner

        lax.fori_loop(0, _L // _JT, jt_body, 0)
        pltpu.sync_copy(contrib, out3.at[ind0b.at[g]], add=True)
        return carry

    lax.fori_loop(0, _GROUPS, group_body, 0)
    plsc.subcore_barrier()
    pltpu.sync_copy(out3.at[pl.ds(s * 32, 32)],
                    out_hbm.at[c, pl.ds(s * 32, 32)])


_KERNEL = pl.kernel(
    _sc_body,
    out_type=jax.ShapeDtypeStruct((2, _NB, 128), jnp.float32),
    mesh=plsc.VectorSubcoreMesh(core_axis_name="c", subcore_axis_name="s"),
    compiler_params=pltpu.CompilerParams(needs_layout_passes=False),
    scratch_types=[
        pltpu.VMEM((_NB * 128,), jnp.float32),
        pltpu.VMEM((_L * 256,), jnp.float32),
        pltpu.VMEM((_L, 128), jnp.float32),
        pltpu.VMEM((_EDGES_PER_TILE,), jnp.int32),
        pltpu.VMEM((_GROUPS, _L), jnp.int32),
        pltpu.VMEM((_L * _B * _L,), jnp.float32),
        pltpu.VMEM_SHARED((_NB, 128), jnp.float32),
    ],
)


@jax.jit
def kernel(x, values, b, rows, cols):
    ind0 = (rows[:: _L * _L] // _L).astype(jnp.int32)
    ind1 = (cols[:: _L * _L] // _L).astype(jnp.int32)
    ind0g = ind0.reshape(_T // _L, _L)

    x3 = x.reshape(_B, _NB, _L).transpose(1, 2, 0).reshape(-1)

    b0 = jnp.broadcast_to(b.reshape(_NB, 1, _L), (_NB, _B, _L)).reshape(_NB, 128)
    binit = jnp.stack([b0, jnp.zeros_like(b0)])

    outp = _KERNEL(x3, values, ind1, ind0g, binit)
    out = outp[0] + outp[1]
    out = out.reshape(_NB, _B, _L).transpose(1, 0, 2).reshape(_B, _NB * _L, 1)
    return out


# speedup vs baseline: 104.0426x; 1.0673x over previous
import jax
import jax.numpy as jnp
from jax import lax
from jax.experimental import pallas as pl
from jax.experimental.pallas import tpu as pltpu
from jax.experimental.pallas import tpu_sc as plsc

_T = 8192
_NB = 512
_B = 8
_L = 16
_NTILES = 32
_EDGES_PER_TILE = _T // _NTILES
_GROUPS = _EDGES_PER_TILE // _L
_JT = 4
_VCHUNK = _L * 256


def _sc_body(x3_hbm, vals_hbm, ind1_hbm, ind0g_hbm, binit_hbm, out_hbm,
             x3l, vbufA, vbufB, xstage, contrib, ind1b, ind0b,
             out3, semv0, semv1):
    c = lax.axis_index("c")
    s = lax.axis_index("s")
    wid = c * 16 + s
    semv = (semv0, semv1)
    vbufs = (vbufA, vbufB)

    pltpu.sync_copy(x3_hbm, x3l)
    pltpu.sync_copy(ind1_hbm.at[pl.ds(wid * _EDGES_PER_TILE, _EDGES_PER_TILE)],
                    ind1b)
    pltpu.sync_copy(ind0g_hbm.at[pl.ds(wid * _GROUPS, _GROUPS)], ind0b)
    pltpu.sync_copy(binit_hbm.at[c, pl.ds(s * 32, 32)], out3.at[pl.ds(s * 32, 32)])
    plsc.subcore_barrier()

    iota = lax.iota(jnp.int32, _L)
    iota_v = iota * 256

    def vals_issue(g, sub):
        base_e = wid * _EDGES_PER_TILE + g * _L
        pltpu.async_copy(vals_hbm.at[pl.ds(base_e * 256, _VCHUNK)],
                         vbufs[sub], semv[sub])

    def vals_wait(sub):
        pltpu.make_async_copy(vals_hbm.at[pl.ds(0, _VCHUNK)],
                              vbufs[sub], semv[sub]).wait()

    def compute_group(g, sub):
        vals_wait(sub)
        vbuf = vbufs[sub]
        ind1v = ind1b[pl.ds(g * _L, _L)]
        rowb = ind1v * 128

        for i in range(_L):
            for bb in range(_B):
                cc = i * _B + bb
                xv = plsc.load_gather(x3l, [rowb + cc])
                xstage[pl.ds(cc * _L, _L)] = xv

        def jt_body(jt, inner):
            jbase = jt * _JT
            acc = [[None] * _B for _ in range(_JT)]
            for i in range(_L):
                vv = [plsc.load_gather(vbuf, [iota_v + (i * 16 + jbase + jp)])
                      for jp in range(_JT)]
                xv = [xstage[pl.ds((i * _B + bb) * _L, _L)]
                      for bb in range(_B)]
                for jp in range(_JT):
                    for bb in range(_B):
                        prod = vv[jp] * xv[bb]
                        if i == 0:
                            acc[jp][bb] = prod
                        else:
                            acc[jp][bb] = acc[jp][bb] + prod
            for jp in range(_JT):
                for bb in range(_B):
                    col = bb * 16 + jbase + jp
                    plsc.store_scatter(contrib,
                                       [iota, jnp.broadcast_to(col, (_L,))],
                                       acc[jp][bb])
            return inner

        lax.fori_loop(0, _L // _JT, jt_body, 0)

        gnext = jnp.minimum(g + 2, _GROUPS - 1)
        vals_issue(gnext, sub)

        pltpu.sync_copy(contrib, out3.at[ind0b.at[g]], add=True)

    vals_issue(jnp.int32(0), 0)
    vals_issue(jnp.int32(1), 1)

    def pair_body(p, carry):
        compute_group(2 * p, 0)
        compute_group(2 * p + 1, 1)
        return carry

    lax.fori_loop(0, _GROUPS // 2, pair_body, 0)

    vals_wait(0)
    vals_wait(1)

    plsc.subcore_barrier()
    pltpu.sync_copy(out3.at[pl.ds(s * 32, 32)],
                    out_hbm.at[c, pl.ds(s * 32, 32)])


_KERNEL = pl.kernel(
    _sc_body,
    out_type=jax.ShapeDtypeStruct((2, _NB, 128), jnp.float32),
    mesh=plsc.VectorSubcoreMesh(core_axis_name="c", subcore_axis_name="s"),
    compiler_params=pltpu.CompilerParams(needs_layout_passes=False),
    scratch_types=[
        pltpu.VMEM((_NB * 128,), jnp.float32),
        pltpu.VMEM((_VCHUNK,), jnp.float32),
        pltpu.VMEM((_VCHUNK,), jnp.float32),
        pltpu.VMEM((_L * _B * _L,), jnp.float32),
        pltpu.VMEM((_L, 128), jnp.float32),
        pltpu.VMEM((_EDGES_PER_TILE,), jnp.int32),
        pltpu.VMEM((_GROUPS, _L), jnp.int32),
        pltpu.VMEM_SHARED((_NB, 128), jnp.float32),
        pltpu.SemaphoreType.DMA,
        pltpu.SemaphoreType.DMA,
    ],
)


@jax.jit
def kernel(x, values, b, rows, cols):
    ind0 = (rows[:: _L * _L] // _L).astype(jnp.int32)
    ind1 = (cols[:: _L * _L] // _L).astype(jnp.int32)
    ind0g = ind0.reshape(_T // _L, _L)

    x3 = x.reshape(_B, _NB, _L).transpose(1, 2, 0).reshape(-1)

    b0 = jnp.broadcast_to(b.reshape(_NB, 1, _L), (_NB, _B, _L)).reshape(_NB, 128)
    binit = jnp.stack([b0, jnp.zeros_like(b0)])

    outp = _KERNEL(x3, values, ind1, ind0g, binit)
    out = outp[0] + outp[1]
    out = out.reshape(_NB, _B, _L).transpose(1, 0, 2).reshape(_B, _NB * _L, 1)
    return out
